# baseline (speedup 1.0000x reference)
import jax
import jax.numpy as jnp
from jax import lax
from jax.experimental import pallas as pl
from jax.experimental.pallas import tpu as pltpu
from jax.experimental.pallas import tpu_sc as plsc

N = 25000
D = 128
E = 400000
NC = 2
NS = 16
DC = D // NC
B = 64
BG = 128
NBLK = 392
NBG = 196
EPT = NBLK * B
EPAD = EPT * NS
SBN = 14
SBB = 28
SBG = 14
NPAD = 25600
FB = 64
NFL = 25
RPT = NFL * FB


_GDN = lax.GatherDimensionNumbers(
    offset_dims=(), collapsed_slice_dims=(0,), start_index_map=(0,))


def _lane_broadcast(v, lane):
    idx = jnp.full((16, 1), lane, jnp.int32)
    return lax.gather(v, idx, dimension_numbers=_GDN, slice_sizes=(1,),
                      mode=lax.GatherScatterMode.PROMISE_IN_BOUNDS)


def _sc_body(eu0, ei0, rows64, cols64, rows128, cols128, vals2,
             zu1, zi1, usum, isum,
             acc, gb0, gb1, sb0, sb1, gidx, sidx2, vbuf, didx,
             g0, g1, s0, s1):
    c = lax.axis_index("c")
    s = lax.axis_index("s")
    iota = lax.iota(jnp.int32, 16)
    zeros16 = jnp.zeros((16,), jnp.float32)
    coff = c * NPAD

    for k in range(B // 16):
        didx[0, pl.ds(k * 16, 16)] = iota + (NPAD - B + k * 16)

    def zero_acc():
        @pl.loop(0, FB, unroll=4)
        def _(r):
            for k in range(DC // 16):
                gb0[r, pl.ds(k * 16, 16)] = zeros16

        base = s * RPT
        zsrc = gb0

        @pl.loop(0, NFL)
        def _(t):
            pltpu.async_copy(zsrc, acc.at[pl.ds(base + t * FB, FB)], g0)

        @pl.loop(0, NFL)
        def _(t):
            pltpu.make_async_copy(
                zsrc, acc.at[pl.ds(base + t * FB, FB)], g0).wait()

    def edge_loop(tbl, g64_hbm, s64_hbm):
        @pl.loop(0, SBN)
        def _(sb):
            rb64 = s * NBLK + sb * SBB
            pltpu.sync_copy(g64_hbm.at[pl.ds(rb64, SBB)], gidx)
            pltpu.sync_copy(s64_hbm.at[pl.ds(rb64, SBB)], sidx2)
            pltpu.sync_copy(vals2.at[pl.ds(rb64, SBB)], vbuf)

            @pl.loop(0, SBB, unroll=2)
            def _(j):
                for k in range(B // 16):
                    ds = pl.ds(k * 16, 16)
                    gidx[j, ds] = gidx[j, ds] + coff

            pltpu.async_copy(tbl.at[gidx.at[0]], gb0, g0)
            pltpu.async_copy(tbl.at[gidx.at[1]], gb1, g1)

            @pl.loop(0, SBB // 2)
            def _(jj):
                for G, SB, gsem, ssem, par in (
                        (gb0, sb0, g0, s0, 0), (gb1, sb1, g1, s1, 1)):
                    b = jj * 2 + par
                    pltpu.make_async_copy(tbl.at[gidx.at[b]], G, gsem).wait()

                    @pl.when(b >= 2)
                    def _():
                        pltpu.make_async_copy(
                            SB, acc.at[sidx2.at[b - 2]], ssem).wait()

                    @pl.loop(0, B // 16)
                    def _(g):
                        vv = vbuf[b, pl.ds(g * 16, 16)]
                        for e16 in range(16):
                            ev = _lane_broadcast(vv, e16)
                            e = g * 16 + e16
                            for k in range(DC // 16):
                                ds = pl.ds(k * 16, 16)
                                SB[e, ds] = G[e, ds] * ev

                    pltpu.async_copy(
                        SB, acc.at[sidx2.at[b]], ssem, add=True)

                    @pl.when(jj < SBB // 2 - 1)
                    def _():
                        pltpu.async_copy(tbl.at[gidx.at[b + 2]], G, gsem)

            pltpu.make_async_copy(sb0, acc.at[sidx2.at[SBB - 2]], s0).wait()
            pltpu.make_async_copy(sb1, acc.at[sidx2.at[SBB - 1]], s1).wait()

    def flush_raw(out):
        buf = gb0

        @pl.loop(0, NFL)
        def _(t):
            r0 = s * RPT + t * FB
            pltpu.sync_copy(acc.at[pl.ds(r0, FB)], buf)
            pltpu.sync_copy(buf, out.at[pl.ds(coff + r0, FB)])

    def flush_combine(e0, z1, out):
        third = jnp.float32(1.0 / 3.0)
        ba = gb0
        bb = gb1

        @pl.loop(0, NFL)
        def _(t):
            r0 = s * RPT + t * FB
            pltpu.sync_copy(acc.at[pl.ds(r0, FB)], ba)
            pltpu.sync_copy(e0.at[pl.ds(coff + r0, FB)], bb)

            @pl.loop(0, FB, unroll=4)
            def _(r):
                for k in range(DC // 16):
                    ds = pl.ds(k * 16, 16)
                    gb0[r, ds] = gb0[r, ds] + gb1[r, ds]

            pltpu.sync_copy(z1.at[pl.ds(coff + r0, FB)], bb)

            @pl.loop(0, FB, unroll=4)
            def _(r):
                for k in range(DC // 16):
                    ds = pl.ds(k * 16, 16)
                    gb0[r, ds] = (gb0[r, ds] + gb1[r, ds]) * third

            pltpu.sync_copy(ba, out.at[pl.ds(coff + r0, FB)])

    zero_acc()
    plsc.subcore_barrier()
    edge_loop(ei0, cols64, rows64)
    plsc.subcore_barrier()
    flush_raw(zu1)
    plsc.subcore_barrier()

    zero_acc()
    plsc.subcore_barrier()
    edge_loop(eu0, rows64, cols64)
    plsc.subcore_barrier()
    flush_raw(zi1)
    plsc.subcore_barrier()

    zero_acc()
    plsc.subcore_barrier()
    edge_loop(zi1, cols64, rows64)
    plsc.subcore_barrier()
    flush_combine(eu0, zu1, usum)
    plsc.subcore_barrier()

    zero_acc()
    plsc.subcore_barrier()
    edge_loop(zu1, rows64, cols64)
    plsc.subcore_barrier()
    flush_combine(ei0, zi1, isum)


_mesh = plsc.VectorSubcoreMesh(
    core_axis_name="c", subcore_axis_name="s", num_cores=NC, num_subcores=NS)

_tbl = jax.ShapeDtypeStruct((NC * NPAD, DC), jnp.float32)

_spmm = pl.kernel(
    _sc_body,
    out_type=(_tbl, _tbl, _tbl, _tbl),
    mesh=_mesh,
    compiler_params=pltpu.CompilerParams(
        needs_layout_passes=False, use_tc_tiling_on_sc=False),
    scratch_types=[
        pltpu.VMEM_SHARED((NPAD, DC), jnp.float32),
        pltpu.VMEM((B, DC), jnp.float32),
        pltpu.VMEM((B, DC), jnp.float32),
        pltpu.VMEM((B, DC), jnp.float32),
        pltpu.VMEM((B, DC), jnp.float32),
        pltpu.VMEM((SBB, B), jnp.int32),
        pltpu.VMEM((SBB, B), jnp.int32),
        pltpu.VMEM((SBB, B), jnp.float32),
        pltpu.VMEM((1, B), jnp.int32),
        pltpu.SemaphoreType.DMA,
        pltpu.SemaphoreType.DMA,
        pltpu.SemaphoreType.DMA,
        pltpu.SemaphoreType.DMA,
    ],
)


def _to_chunked(x):
    xt = x.reshape(N, NC, DC).transpose(1, 0, 2)
    return jnp.pad(xt, ((0, 0), (0, NPAD - N), (0, 0))).reshape(NC * NPAD, DC)


def _from_chunked(x):
    return x.reshape(NC, NPAD, DC)[:, :N].transpose(1, 0, 2).reshape(N, D)


@jax.jit
def kernel(E_u_0, E_i_0, adj_indices, adj_values, u_mul_s, v_mul_s, ut, vt):
    rows = adj_indices[0].astype(jnp.int32)
    cols = adj_indices[1].astype(jnp.int32)
    vals = adj_values.astype(jnp.float32)
    pad = EPAD - E
    rows_p = jnp.concatenate([rows, jnp.zeros((pad,), jnp.int32)])
    cols_p = jnp.concatenate([cols, jnp.zeros((pad,), jnp.int32)])
    vals_p = jnp.concatenate(
        [vals, jnp.zeros((pad,), jnp.float32)]).reshape(EPAD // B, B)
    rows64 = rows_p.reshape(EPAD // B, B)
    cols64 = cols_p.reshape(EPAD // B, B)
    rows128, cols128 = lax.optimization_barrier(
        (rows_p.reshape(EPAD // BG, BG), cols_p.reshape(EPAD // BG, BG)))
    eu0 = _to_chunked(E_u_0)
    ei0 = _to_chunked(E_i_0)
    _, _, us, it = _spmm(eu0, ei0, rows64, cols64, rows128, cols128, vals_p)
    return _from_chunked(us), _from_chunked(it)
